# Initial kernel scaffold; baseline (speedup 1.0000x reference)
import jax
import jax.numpy as jnp
from jax.experimental import pallas as pl


def kernel(x, tables, W1, b1, W2, b2, W3, b3):
    raise NotImplementedError("write your pallas kernel here")


# speedup vs baseline: 2.7744x; 2.7744x over previous
import functools

import jax
import jax.numpy as jnp
from jax import lax
from jax.experimental import pallas as pl
from jax.experimental.pallas import tpu as pltpu
from jax.experimental.pallas import tpu_sc as plsc

_NC = 2
_NS = 16
_NW = _NC * _NS
_CHUNK = 8192


def _sc_gather_t(tab2, xt):
    fd, v = tab2.shape
    f, b = xt.shape
    d = fd // f
    n_jobs = fd // _NW
    mesh = plsc.VectorSubcoreMesh(core_axis_name="c", subcore_axis_name="s")

    @functools.partial(
        pl.kernel,
        out_type=jax.ShapeDtypeStruct((fd, b), jnp.float32),
        mesh=mesh,
        compiler_params=pltpu.CompilerParams(needs_layout_passes=False),
        scratch_types=[
            pltpu.VMEM((v,), jnp.float32),
            pltpu.VMEM((_CHUNK,), jnp.int32),
            pltpu.VMEM((_CHUNK,), jnp.float32),
        ],
    )
    def gather_k(tab_hbm, xt_hbm, out_hbm, row_v, idx_v, outb_v):
        cid = lax.axis_index("c")
        sid = lax.axis_index("s")
        wid = sid * _NC + cid

        def job(j, carry):
            r = j * _NW + wid
            pltpu.sync_copy(tab_hbm.at[r], row_v)

            def chunk(ci, c2):
                col0 = ci * _CHUNK
                pltpu.sync_copy(xt_hbm.at[j, pl.ds(col0, _CHUNK)], idx_v)

                def g(i, c3):
                    sl = pl.ds(i * 16, 16)
                    outb_v[sl] = plsc.load_gather(row_v, [idx_v[sl]])
                    return c3

                lax.fori_loop(0, _CHUNK // 16, g, 0, unroll=8)
                pltpu.sync_copy(outb_v, out_hbm.at[r, pl.ds(col0, _CHUNK)])
                return c2

            lax.fori_loop(0, b // _CHUNK, chunk, 0)
            return carry

        lax.fori_loop(0, n_jobs, job, 0)

    return gather_k(tab2, xt)


def _mlp_body(embt_ref, s_ref, w1_ref, b1_ref, w2_ref, b2_ref, w3_ref, b3_ref,
              out_ref):
    dn = (((0,), (0,)), ((), ()))
    ft = embt_ref[...]
    sp = lax.dot_general(s_ref[...], ft, dn,
                         preferred_element_type=jnp.float32)
    left = jnp.sum(sp * sp, axis=0, keepdims=True)
    right = jnp.sum(ft * ft, axis=0, keepdims=True)
    fm = 0.5 * (left - right)
    h = lax.dot_general(w1_ref[...], ft, dn,
                        preferred_element_type=jnp.float32)
    h = jnp.maximum(h + b1_ref[...], 0.0)
    h = lax.dot_general(w2_ref[...], h, dn,
                        preferred_element_type=jnp.float32)
    h = jnp.maximum(h + b2_ref[...], 0.0)
    h = lax.dot_general(w3_ref[...], h, dn,
                        preferred_element_type=jnp.float32)
    h = jnp.maximum(h + b3_ref[...], 0.0)
    out_ref[...] = fm + h


def _tc_mlp_t(embt, s, w1, b1c, w2p, b2c, w3p, b3c, bb):
    din, b = embt.shape
    grid = (b // bb,)
    return pl.pallas_call(
        _mlp_body,
        grid=grid,
        in_specs=[
            pl.BlockSpec((din, bb), lambda i: (0, i)),
            pl.BlockSpec(s.shape, lambda i: (0, 0)),
            pl.BlockSpec(w1.shape, lambda i: (0, 0)),
            pl.BlockSpec(b1c.shape, lambda i: (0, 0)),
            pl.BlockSpec(w2p.shape, lambda i: (0, 0)),
            pl.BlockSpec(b2c.shape, lambda i: (0, 0)),
            pl.BlockSpec(w3p.shape, lambda i: (0, 0)),
            pl.BlockSpec(b3c.shape, lambda i: (0, 0)),
        ],
        out_specs=pl.BlockSpec((128, bb), lambda i: (0, i)),
        out_shape=jax.ShapeDtypeStruct((128, b), jnp.float32),
    )(embt, s, w1, b1c, w2p, b2c, w3p, b3c)


def kernel(x, tables, W1, b1, W2, b2, W3, b3):
    B, F = x.shape
    V, D = tables.shape[1], tables.shape[2]

    tab2 = tables.transpose(0, 2, 1).reshape(F * D, V)
    xt = x.T.astype(jnp.int32)

    embt = _sc_gather_t(tab2, xt)

    s = jnp.tile(jnp.eye(D, dtype=jnp.float32), (F, 1))
    b1c = b1.reshape(128, 1)
    w2p = jnp.pad(W2, ((0, 0), (0, 128 - W2.shape[1])))
    b2c = jnp.pad(b2, (0, 128 - b2.shape[0])).reshape(128, 1)
    w3p = jnp.pad(W3, ((0, 128 - W3.shape[0]), (0, 128 - W3.shape[1])))
    b3c = jnp.pad(b3, (0, 128 - b3.shape[0])).reshape(128, 1)

    out_t = _tc_mlp_t(embt, s, W1, b1c, w2p, b2c, w3p, b3c, bb=512)
    return out_t[:2, :].T
